# baseline (speedup 1.0000x reference)
import functools

import jax
import jax.numpy as jnp
from jax import lax
from jax.experimental import pallas as pl
from jax.experimental.pallas import tpu as pltpu
from jax.experimental.pallas import tpu_sc as plsc

_B = 16384
_D = 32
_K = 5
_V = 1000000
_CLIP = 10.0

_NC = 2
_NS = 16
_L = 16
_NW = _NC * _NS
_BW = _B // _NW
_G = 32
_NG = _BW // _G
_R4 = 250112
_S1, _S2, _S3 = 250112, 500224, 750336
_RB = 256
_GRID = _R4 // _RB


def _transpose_body(t0, t1, t2, t3, out):
  out[...] = jnp.concatenate(
      [t0[...].T, t1[...].T, t2[...].T, t3[...].T], axis=1)


def _pack_table(table_t):
  nblk = _V // _RB
  in_specs = [
      pl.BlockSpec(
          (_D, _RB),
          functools.partial(
              lambda j, g: (0, jnp.minimum((_S1 // _RB) * j + g, nblk)), j))
      for j in range(4)
  ]
  return pl.pallas_call(
      _transpose_body,
      grid=(_GRID,),
      in_specs=in_specs,
      out_specs=pl.BlockSpec((_RB, 4 * _D), lambda g: (g, 0)),
      out_shape=jax.ShapeDtypeStruct((_R4, 4 * _D), jnp.float32),
  )(table_t, table_t, table_t, table_t)


def _t4row(i):
  j = ((i >= _S1).astype(jnp.int32) + (i >= _S2).astype(jnp.int32)
       + (i >= _S3).astype(jnp.int32))
  return i - _S1 * j, _D * j


def _sc_scores(center_ids, context_ids, neg_ids_flat, t4c, t4x):
  mesh = plsc.VectorSubcoreMesh(core_axis_name="c", subcore_axis_name="s")

  @functools.partial(
      pl.kernel,
      mesh=mesh,
      compiler_params=pltpu.CompilerParams(
          needs_layout_passes=False, use_tc_tiling_on_sc=True),
      out_type=[
          jax.ShapeDtypeStruct((_B,), jnp.float32),
          jax.ShapeDtypeStruct((_B * _K,), jnp.float32),
      ],
      scratch_types=[
          pltpu.VMEM((_BW,), jnp.int32),
          pltpu.VMEM((_BW,), jnp.int32),
          pltpu.VMEM((_BW * _K,), jnp.int32),
          pltpu.VMEM((_BW,), jnp.int32),
          pltpu.VMEM((_BW,), jnp.int32),
          pltpu.VMEM((_K * _BW,), jnp.int32),
          pltpu.VMEM((2, _G, 4 * _D), jnp.float32),
          pltpu.VMEM((2, _G, 4 * _D), jnp.float32),
          pltpu.VMEM((2, _K * _G, 4 * _D), jnp.float32),
          pltpu.VMEM((_BW,), jnp.float32),
          pltpu.VMEM((_K * _BW,), jnp.float32),
          pltpu.SemaphoreType.DMA,
          pltpu.SemaphoreType.DMA,
      ],
  )
  def body(cid_hbm, xid_hbm, nid_hbm, t4c_hbm, t4x_hbm,
           pos_hbm, neg_hbm,
           idx_c, idx_x, idx_n, row_c, row_x, row_n,
           st_c, st_x, st_n, pos_v, neg_v, sem0, sem1):
    wid = lax.axis_index("s") * _NC + lax.axis_index("c")
    base = wid * _BW
    nbase = wid * (_BW * _K)

    pltpu.sync_copy(cid_hbm.at[pl.ds(base, _BW)], idx_c)
    pltpu.sync_copy(xid_hbm.at[pl.ds(base, _BW)], idx_x)
    pltpu.sync_copy(nid_hbm.at[pl.ds(nbase, _BW * _K)], idx_n)

    iota = lax.iota(jnp.int32, _L)

    def rows_body(v, _):
      b_vec = v * _L + iota
      sl = pl.ds(v * _L, _L)
      r, _cj = _t4row(idx_c[sl])
      row_c[sl] = r
      r, _cj = _t4row(idx_x[sl])
      row_x[sl] = r
      for k in range(_K):
        nid = plsc.load_gather(idx_n, [b_vec * _K + k])
        r, _cj = _t4row(nid)
        row_n[pl.ds(k * _BW + v * _L, _L)] = r
      return _

    lax.fori_loop(0, _BW // _L, rows_body, 0)

    sems = (sem0, sem1)

    def fire(g, buf):
      sem = sems[buf]
      gsl = pl.ds(g * _G, _G)
      pltpu.async_copy(t4c_hbm.at[row_c.at[gsl]], st_c.at[buf], sem)
      pltpu.async_copy(t4x_hbm.at[row_x.at[gsl]], st_x.at[buf], sem)
      for k in range(_K):
        pltpu.async_copy(
            t4x_hbm.at[row_n.at[pl.ds(k * _BW + g * _G, _G)]],
            st_n.at[buf, pl.ds(k * _G, _G)], sem)

    def drain(buf):
      sem = sems[buf]
      pltpu.make_async_copy(t4c_hbm.at[pl.ds(0, _G)], st_c.at[buf], sem).wait()
      pltpu.make_async_copy(t4c_hbm.at[pl.ds(0, _G)], st_x.at[buf], sem).wait()
      pltpu.make_async_copy(
          t4c_hbm.at[pl.ds(0, _K * _G)], st_n.at[buf], sem).wait()

    def compute(g, buf):
      for h in range(_G // _L):
        loc = h * _L + iota
        sl = pl.ds(g * _G + h * _L, _L)
        _r, cj_c = _t4row(idx_c[sl])
        _r, cj_x = _t4row(idx_x[sl])
        acc_p = jnp.zeros((_L,), jnp.float32)
        accs = []
        cjs = []
        for k in range(_K):
          nid = plsc.load_gather(idx_n, [(g * _G + loc) * _K + k])
          _r, cj_n = _t4row(nid)
          cjs.append(cj_n)
          accs.append(jnp.zeros((_L,), jnp.float32))
        for d in range(_D):
          c_col = plsc.load_gather(st_c, [jnp.full((_L,), buf, jnp.int32),
                                          loc, cj_c + d])
          x_col = plsc.load_gather(st_x, [jnp.full((_L,), buf, jnp.int32),
                                          loc, cj_x + d])
          acc_p = acc_p + c_col * x_col
          for k in range(_K):
            n_col = plsc.load_gather(
                st_n, [jnp.full((_L,), buf, jnp.int32),
                       k * _G + loc, cjs[k] + d])
            accs[k] = accs[k] + c_col * n_col
        pos_v[sl] = acc_p
        for k in range(_K):
          neg_v[pl.ds(k * _BW + g * _G + h * _L, _L)] = accs[k]

    fire(0, 0)

    def pipe_body(it, _):
      g0 = it * 2
      fire(g0 + 1, 1)
      drain(0)
      compute(g0, 0)

      @pl.when(g0 + 2 < _NG)
      def _fire_next():
        fire(g0 + 2, 0)

      drain(1)
      compute(g0 + 1, 1)
      return _

    lax.fori_loop(0, _NG // 2, pipe_body, 0)

    pltpu.sync_copy(pos_v, pos_hbm.at[pl.ds(base, _BW)])
    pltpu.sync_copy(neg_v, neg_hbm.at[pl.ds(nbase, _BW * _K)])

  return body(center_ids, context_ids, neg_ids_flat, t4c, t4x)


def _finish_body(pos_ref, neg_ref, out_ref):
  p = jnp.clip(pos_ref[...], -_CLIP, _CLIP)
  n = jnp.clip(neg_ref[...], -_CLIP, _CLIP)
  lp = jnp.maximum(-p, 0.0) + jnp.log1p(jnp.exp(-jnp.abs(p)))
  ln = jnp.maximum(n, 0.0) + jnp.log1p(jnp.exp(-jnp.abs(n)))
  total = jnp.sum(lp) + jnp.sum(ln)
  out_ref[...] = jnp.reshape(total * (1.0 / _B), (1, 1))


def kernel(center_ids, context_ids, neg_context_ids, center_emb, context_emb):
  cid = center_ids.astype(jnp.int32)
  xid = context_ids.astype(jnp.int32)
  nid = neg_context_ids.astype(jnp.int32).reshape(-1)
  t4c = _pack_table(center_emb.T)
  t4x = _pack_table(context_emb.T)
  pos, neg = _sc_scores(cid, xid, nid, t4c, t4x)
  out = pl.pallas_call(
      _finish_body,
      out_shape=jax.ShapeDtypeStruct((1, 1), jnp.float32),
  )(pos.reshape(_B // 128, 128), neg.reshape(_B * _K // 128, 128))
  return out[0, 0]


# speedup vs baseline: 1.0566x; 1.0566x over previous
import functools

import jax
import jax.numpy as jnp
from jax import lax
from jax.experimental import pallas as pl
from jax.experimental.pallas import tpu as pltpu
from jax.experimental.pallas import tpu_sc as plsc

_B = 16384
_D = 32
_K = 5
_V = 1000000
_CLIP = 10.0

_NC = 2
_NS = 16
_L = 16
_NW = _NC * _NS
_BW = _B // _NW
_G = 32
_NG = _BW // _G
_R4 = 250112
_S1, _S2, _S3 = 250112, 500224, 750336
_RB = 256
_GRID = _R4 // _RB


_W = 128
_NWIN = _R4 // _W
_CLAMP3 = 999936


def _sc_pack(ttc, ttx):
  mesh = plsc.VectorSubcoreMesh(core_axis_name="c", subcore_axis_name="s")

  @functools.partial(
      pl.kernel,
      mesh=mesh,
      compiler_params=pltpu.CompilerParams(
          needs_layout_passes=False, use_tc_tiling_on_sc=True),
      out_type=[
          jax.ShapeDtypeStruct((_R4, 4 * _D), jnp.float32),
          jax.ShapeDtypeStruct((_R4, 4 * _D), jnp.float32),
      ],
      scratch_types=[
          pltpu.VMEM((2, 4, _D, _W), jnp.float32),
          pltpu.VMEM((2, _W, 4 * _D), jnp.float32),
          pltpu.SemaphoreType.DMA,
          pltpu.SemaphoreType.DMA,
          pltpu.SemaphoreType.DMA,
          pltpu.SemaphoreType.DMA,
      ],
  )
  def body(tc_hbm, tx_hbm, t4c_hbm, t4x_hbm, in_v, out_v, si0, si1, so0, so1):
    wid = lax.axis_index("s") * _NC + lax.axis_index("c")
    iota = lax.iota(jnp.int32, _L)
    nt = jnp.where(wid < _NWIN - 32 * (_NWIN // 32), 1 + _NWIN // 32,
                   _NWIN // 32)

    for (src, dst) in ((tc_hbm, t4c_hbm), (tx_hbm, t4x_hbm)):

      def offs(g):
        o = [_S1 * j + _W * g for j in range(3)]
        o.append(jnp.minimum(_S1 * 3 + _W * g, _CLAMP3))
        return o

      def fire_in_p(g, b, sem):
        for j, o in enumerate(offs(g)):
          pltpu.async_copy(src.at[:, pl.ds(o, _W)], in_v.at[b, j], sem)

      def drain_in_p(b, sem):
        for j in range(4):
          pltpu.make_async_copy(src.at[:, pl.ds(0, _W)], in_v.at[b, j],
                                sem).wait()

      def compute(g, b):
        bvec = jnp.full((_L,), 0, jnp.int32) + b
        for rc in range(_W // _L):
          rvec = rc * _L + iota

          def dbody(d, _):
            for j in range(4):
              v = in_v[b, j, d, pl.ds(rc * _L, _L)]
              plsc.store_scatter(out_v, [bvec, rvec,
                                         jnp.full((_L,), _D * j, jnp.int32) + d],
                                 v)
            return _
          lax.fori_loop(0, _D, dbody, 0)

      def fire_out_p(g, b, sem):
        pltpu.async_copy(out_v.at[b], dst.at[pl.ds(_W * g, _W)], sem)

      def drain_out_p(b, sem):
        pltpu.make_async_copy(out_v.at[b], dst.at[pl.ds(0, _W)], sem).wait()

      fire_in_p(wid, 0, si0)

      def wbody(t, _):
        g = wid + 32 * t
        b = t % 2

        @pl.when(t + 1 < nt)
        def _pref():
          gn = g + 32
          bn = (t + 1) % 2

          @pl.when(bn == 0)
          def _f0():
            fire_in_p(gn, 0, si0)

          @pl.when(bn == 1)
          def _f1():
            fire_in_p(gn, 1, si1)

        @pl.when(b == 0)
        def _w0():
          drain_in_p(0, si0)

          @pl.when(t >= 2)
          def _d0():
            drain_out_p(0, so0)
          compute(g, 0)
          fire_out_p(g, 0, so0)

        @pl.when(b == 1)
        def _w1():
          drain_in_p(1, si1)

          @pl.when(t >= 2)
          def _d1():
            drain_out_p(1, so1)
          compute(g, 1)
          fire_out_p(g, 1, so1)
        return _

      lax.fori_loop(0, nt, wbody, 0)
      drain_out_p(0, so0)
      drain_out_p(1, so1)

  return body(ttc, ttx)


def _t4row(i):
  j = ((i >= _S1).astype(jnp.int32) + (i >= _S2).astype(jnp.int32)
       + (i >= _S3).astype(jnp.int32))
  return i - _S1 * j, _D * j


def _sc_scores(center_ids, context_ids, neg_ids_flat, t4c, t4x):
  mesh = plsc.VectorSubcoreMesh(core_axis_name="c", subcore_axis_name="s")

  @functools.partial(
      pl.kernel,
      mesh=mesh,
      compiler_params=pltpu.CompilerParams(
          needs_layout_passes=False, use_tc_tiling_on_sc=True),
      out_type=[
          jax.ShapeDtypeStruct((_B,), jnp.float32),
          jax.ShapeDtypeStruct((_B * _K,), jnp.float32),
      ],
      scratch_types=[
          pltpu.VMEM((_BW,), jnp.int32),
          pltpu.VMEM((_BW,), jnp.int32),
          pltpu.VMEM((_BW * _K,), jnp.int32),
          pltpu.VMEM((_BW,), jnp.int32),
          pltpu.VMEM((_BW,), jnp.int32),
          pltpu.VMEM((_K * _BW,), jnp.int32),
          pltpu.VMEM((2, _G, 4 * _D), jnp.float32),
          pltpu.VMEM((2, _G, 4 * _D), jnp.float32),
          pltpu.VMEM((2, _K * _G, 4 * _D), jnp.float32),
          pltpu.VMEM((_BW,), jnp.float32),
          pltpu.VMEM((_K * _BW,), jnp.float32),
          pltpu.SemaphoreType.DMA,
          pltpu.SemaphoreType.DMA,
      ],
  )
  def body(cid_hbm, xid_hbm, nid_hbm, t4c_hbm, t4x_hbm,
           pos_hbm, neg_hbm,
           idx_c, idx_x, idx_n, row_c, row_x, row_n,
           st_c, st_x, st_n, pos_v, neg_v, sem0, sem1):
    wid = lax.axis_index("s") * _NC + lax.axis_index("c")
    base = wid * _BW
    nbase = wid * (_BW * _K)

    pltpu.sync_copy(cid_hbm.at[pl.ds(base, _BW)], idx_c)
    pltpu.sync_copy(xid_hbm.at[pl.ds(base, _BW)], idx_x)
    pltpu.sync_copy(nid_hbm.at[pl.ds(nbase, _BW * _K)], idx_n)

    iota = lax.iota(jnp.int32, _L)

    def rows_body(v, _):
      b_vec = v * _L + iota
      sl = pl.ds(v * _L, _L)
      r, _cj = _t4row(idx_c[sl])
      row_c[sl] = r
      r, _cj = _t4row(idx_x[sl])
      row_x[sl] = r
      for k in range(_K):
        nid = plsc.load_gather(idx_n, [b_vec * _K + k])
        r, _cj = _t4row(nid)
        row_n[pl.ds(k * _BW + v * _L, _L)] = r
      return _

    lax.fori_loop(0, _BW // _L, rows_body, 0)

    sems = (sem0, sem1)

    def fire(g, buf):
      sem = sems[buf]
      gsl = pl.ds(g * _G, _G)
      pltpu.async_copy(t4c_hbm.at[row_c.at[gsl]], st_c.at[buf], sem)
      pltpu.async_copy(t4x_hbm.at[row_x.at[gsl]], st_x.at[buf], sem)
      for k in range(_K):
        pltpu.async_copy(
            t4x_hbm.at[row_n.at[pl.ds(k * _BW + g * _G, _G)]],
            st_n.at[buf, pl.ds(k * _G, _G)], sem)

    def drain(buf):
      sem = sems[buf]
      pltpu.make_async_copy(t4c_hbm.at[pl.ds(0, _G)], st_c.at[buf], sem).wait()
      pltpu.make_async_copy(t4c_hbm.at[pl.ds(0, _G)], st_x.at[buf], sem).wait()
      pltpu.make_async_copy(
          t4c_hbm.at[pl.ds(0, _K * _G)], st_n.at[buf], sem).wait()

    def compute(g, buf):
      for h in range(_G // _L):
        loc = h * _L + iota
        sl = pl.ds(g * _G + h * _L, _L)
        _r, cj_c = _t4row(idx_c[sl])
        _r, cj_x = _t4row(idx_x[sl])
        acc_p = jnp.zeros((_L,), jnp.float32)
        accs = []
        cjs = []
        for k in range(_K):
          nid = plsc.load_gather(idx_n, [(g * _G + loc) * _K + k])
          _r, cj_n = _t4row(nid)
          cjs.append(cj_n)
          accs.append(jnp.zeros((_L,), jnp.float32))
        for d in range(_D):
          c_col = plsc.load_gather(st_c, [jnp.full((_L,), buf, jnp.int32),
                                          loc, cj_c + d])
          x_col = plsc.load_gather(st_x, [jnp.full((_L,), buf, jnp.int32),
                                          loc, cj_x + d])
          acc_p = acc_p + c_col * x_col
          for k in range(_K):
            n_col = plsc.load_gather(
                st_n, [jnp.full((_L,), buf, jnp.int32),
                       k * _G + loc, cjs[k] + d])
            accs[k] = accs[k] + c_col * n_col
        pos_v[sl] = acc_p
        for k in range(_K):
          neg_v[pl.ds(k * _BW + g * _G + h * _L, _L)] = accs[k]

    fire(0, 0)

    def pipe_body(it, _):
      g0 = it * 2
      fire(g0 + 1, 1)
      drain(0)
      compute(g0, 0)

      @pl.when(g0 + 2 < _NG)
      def _fire_next():
        fire(g0 + 2, 0)

      drain(1)
      compute(g0 + 1, 1)
      return _

    lax.fori_loop(0, _NG // 2, pipe_body, 0)

    pltpu.sync_copy(pos_v, pos_hbm.at[pl.ds(base, _BW)])
    pltpu.sync_copy(neg_v, neg_hbm.at[pl.ds(nbase, _BW * _K)])

  return body(center_ids, context_ids, neg_ids_flat, t4c, t4x)


def _finish_body(pos_ref, neg_ref, out_ref):
  p = jnp.clip(pos_ref[...], -_CLIP, _CLIP)
  n = jnp.clip(neg_ref[...], -_CLIP, _CLIP)
  lp = jnp.maximum(-p, 0.0) + jnp.log1p(jnp.exp(-jnp.abs(p)))
  ln = jnp.maximum(n, 0.0) + jnp.log1p(jnp.exp(-jnp.abs(n)))
  total = jnp.sum(lp) + jnp.sum(ln)
  out_ref[...] = jnp.reshape(total * (1.0 / _B), (1, 1))


def kernel(center_ids, context_ids, neg_context_ids, center_emb, context_emb):
  cid = center_ids.astype(jnp.int32)
  xid = context_ids.astype(jnp.int32)
  nid = neg_context_ids.astype(jnp.int32).reshape(-1)
  t4c, t4x = _sc_pack(center_emb.T, context_emb.T)
  pos, neg = _sc_scores(cid, xid, nid, t4c, t4x)
  out = pl.pallas_call(
      _finish_body,
      out_shape=jax.ShapeDtypeStruct((1, 1), jnp.float32),
  )(pos.reshape(_B // 128, 128), neg.reshape(_B * _K // 128, 128))
  return out[0, 0]


# speedup vs baseline: 1.4739x; 1.3949x over previous
import functools

import jax
import jax.numpy as jnp
from jax import lax
from jax.experimental import pallas as pl
from jax.experimental.pallas import tpu as pltpu
from jax.experimental.pallas import tpu_sc as plsc

_B = 16384
_D = 32
_K = 5
_V = 1000000
_CLIP = 10.0

_NC = 2
_NS = 16
_L = 16
_NW = _NC * _NS
_BW = _B // _NW
_G = 32
_NG = _BW // _G
_R4 = 250112
_S1, _S2, _S3 = 250112, 500224, 750336
_RB = 256
_GRID = _R4 // _RB


_W = 128
_NWIN = _R4 // _W
_CLAMP3 = 999936


def _sc_pack(ttc, ttx):
  mesh = plsc.VectorSubcoreMesh(core_axis_name="c", subcore_axis_name="s")

  @functools.partial(
      pl.kernel,
      mesh=mesh,
      compiler_params=pltpu.CompilerParams(
          needs_layout_passes=False, use_tc_tiling_on_sc=True),
      out_type=[
          jax.ShapeDtypeStruct((_R4, 4 * _D), jnp.float32),
          jax.ShapeDtypeStruct((_R4, 4 * _D), jnp.float32),
      ],
      scratch_types=[
          pltpu.VMEM((2, 4, _D, _W), jnp.float32),
          pltpu.VMEM((2, _W, 4 * _D), jnp.float32),
          pltpu.SemaphoreType.DMA,
          pltpu.SemaphoreType.DMA,
          pltpu.SemaphoreType.DMA,
          pltpu.SemaphoreType.DMA,
      ],
  )
  def body(tc_hbm, tx_hbm, t4c_hbm, t4x_hbm, in_v, out_v, si0, si1, so0, so1):
    wid = lax.axis_index("s") * _NC + lax.axis_index("c")
    iota = lax.iota(jnp.int32, _L)
    nt = jnp.where(wid < _NWIN - 32 * (_NWIN // 32), 1 + _NWIN // 32,
                   _NWIN // 32)

    for (src, dst) in ((tc_hbm, t4c_hbm), (tx_hbm, t4x_hbm)):

      def offs(g):
        o = [_S1 * j + _W * g for j in range(3)]
        o.append(jnp.minimum(_S1 * 3 + _W * g, _CLAMP3))
        return o

      def fire_in_p(g, b, sem):
        for j, o in enumerate(offs(g)):
          pltpu.async_copy(src.at[:, pl.ds(o, _W)], in_v.at[b, j], sem)

      def drain_in_p(b, sem):
        for j in range(4):
          pltpu.make_async_copy(src.at[:, pl.ds(0, _W)], in_v.at[b, j],
                                sem).wait()

      def compute(g, b):
        bvec = jnp.full((_L,), 0, jnp.int32) + b
        for rc in range(_W // _L):
          rvec = rc * _L + iota

          def dbody(d, _):
            for j in range(4):
              v = in_v[b, j, d, pl.ds(rc * _L, _L)]
              plsc.store_scatter(out_v, [bvec, rvec,
                                         jnp.full((_L,), _D * j, jnp.int32) + d],
                                 v)
            return _
          lax.fori_loop(0, _D, dbody, 0)

      def fire_out_p(g, b, sem):
        pltpu.async_copy(out_v.at[b], dst.at[pl.ds(_W * g, _W)], sem)

      def drain_out_p(b, sem):
        pltpu.make_async_copy(out_v.at[b], dst.at[pl.ds(0, _W)], sem).wait()

      fire_in_p(wid, 0, si0)

      def wbody(t, _):
        g = wid + 32 * t
        b = t % 2

        @pl.when(t + 1 < nt)
        def _pref():
          gn = g + 32
          bn = (t + 1) % 2

          @pl.when(bn == 0)
          def _f0():
            fire_in_p(gn, 0, si0)

          @pl.when(bn == 1)
          def _f1():
            fire_in_p(gn, 1, si1)

        @pl.when(b == 0)
        def _w0():
          drain_in_p(0, si0)

          @pl.when(t >= 2)
          def _d0():
            drain_out_p(0, so0)
          compute(g, 0)
          fire_out_p(g, 0, so0)

        @pl.when(b == 1)
        def _w1():
          drain_in_p(1, si1)

          @pl.when(t >= 2)
          def _d1():
            drain_out_p(1, so1)
          compute(g, 1)
          fire_out_p(g, 1, so1)
        return _

      lax.fori_loop(0, nt, wbody, 0)
      drain_out_p(0, so0)
      drain_out_p(1, so1)

  return body(ttc, ttx)


def _t4row(i):
  return lax.shift_right_logical(i, 2), _D * (i & 3)


def _sc_scores(center_ids, context_ids, neg_ids_flat, t4c, t4x):
  mesh = plsc.VectorSubcoreMesh(core_axis_name="c", subcore_axis_name="s")

  @functools.partial(
      pl.kernel,
      mesh=mesh,
      compiler_params=pltpu.CompilerParams(
          needs_layout_passes=False, use_tc_tiling_on_sc=True),
      out_type=[
          jax.ShapeDtypeStruct((_B,), jnp.float32),
          jax.ShapeDtypeStruct((_B * _K,), jnp.float32),
      ],
      scratch_types=[
          pltpu.VMEM((_BW,), jnp.int32),
          pltpu.VMEM((_BW,), jnp.int32),
          pltpu.VMEM((_BW * _K,), jnp.int32),
          pltpu.VMEM((_BW,), jnp.int32),
          pltpu.VMEM((_BW,), jnp.int32),
          pltpu.VMEM((_K * _BW,), jnp.int32),
          pltpu.VMEM((2, _G, 4 * _D), jnp.float32),
          pltpu.VMEM((2, _G, 4 * _D), jnp.float32),
          pltpu.VMEM((2, _K * _G, 4 * _D), jnp.float32),
          pltpu.VMEM((_BW,), jnp.float32),
          pltpu.VMEM((_K * _BW,), jnp.float32),
          pltpu.SemaphoreType.DMA,
          pltpu.SemaphoreType.DMA,
      ],
  )
  def body(cid_hbm, xid_hbm, nid_hbm, t4c_hbm, t4x_hbm,
           pos_hbm, neg_hbm,
           idx_c, idx_x, idx_n, row_c, row_x, row_n,
           st_c, st_x, st_n, pos_v, neg_v, sem0, sem1):
    wid = lax.axis_index("s") * _NC + lax.axis_index("c")
    base = wid * _BW
    nbase = wid * (_BW * _K)

    pltpu.sync_copy(cid_hbm.at[pl.ds(base, _BW)], idx_c)
    pltpu.sync_copy(xid_hbm.at[pl.ds(base, _BW)], idx_x)
    pltpu.sync_copy(nid_hbm.at[pl.ds(nbase, _BW * _K)], idx_n)

    iota = lax.iota(jnp.int32, _L)

    def rows_body(v, _):
      b_vec = v * _L + iota
      sl = pl.ds(v * _L, _L)
      r, _cj = _t4row(idx_c[sl])
      row_c[sl] = r
      r, _cj = _t4row(idx_x[sl])
      row_x[sl] = r
      for k in range(_K):
        nid = plsc.load_gather(idx_n, [b_vec * _K + k])
        r, _cj = _t4row(nid)
        row_n[pl.ds(k * _BW + v * _L, _L)] = r
      return _

    lax.fori_loop(0, _BW // _L, rows_body, 0)

    sems = (sem0, sem1)

    def fire(g, buf):
      sem = sems[buf]
      gsl = pl.ds(g * _G, _G)
      pltpu.async_copy(t4c_hbm.at[row_c.at[gsl]], st_c.at[buf], sem)
      pltpu.async_copy(t4x_hbm.at[row_x.at[gsl]], st_x.at[buf], sem)
      for k in range(_K):
        pltpu.async_copy(
            t4x_hbm.at[row_n.at[pl.ds(k * _BW + g * _G, _G)]],
            st_n.at[buf, pl.ds(k * _G, _G)], sem)

    def drain(buf):
      sem = sems[buf]
      pltpu.make_async_copy(t4c_hbm.at[pl.ds(0, _G)], st_c.at[buf], sem).wait()
      pltpu.make_async_copy(t4c_hbm.at[pl.ds(0, _G)], st_x.at[buf], sem).wait()
      pltpu.make_async_copy(
          t4c_hbm.at[pl.ds(0, _K * _G)], st_n.at[buf], sem).wait()

    def compute(g, buf):
      for h in range(_G // _L):
        loc = h * _L + iota
        sl = pl.ds(g * _G + h * _L, _L)
        _r, cj_c = _t4row(idx_c[sl])
        _r, cj_x = _t4row(idx_x[sl])
        acc_p = jnp.zeros((_L,), jnp.float32)
        accs = []
        cjs = []
        for k in range(_K):
          nid = plsc.load_gather(idx_n, [(g * _G + loc) * _K + k])
          _r, cj_n = _t4row(nid)
          cjs.append(cj_n)
          accs.append(jnp.zeros((_L,), jnp.float32))
        for d in range(_D):
          c_col = plsc.load_gather(st_c, [jnp.full((_L,), buf, jnp.int32),
                                          loc, cj_c + d])
          x_col = plsc.load_gather(st_x, [jnp.full((_L,), buf, jnp.int32),
                                          loc, cj_x + d])
          acc_p = acc_p + c_col * x_col
          for k in range(_K):
            n_col = plsc.load_gather(
                st_n, [jnp.full((_L,), buf, jnp.int32),
                       k * _G + loc, cjs[k] + d])
            accs[k] = accs[k] + c_col * n_col
        pos_v[sl] = acc_p
        for k in range(_K):
          neg_v[pl.ds(k * _BW + g * _G + h * _L, _L)] = accs[k]

    fire(0, 0)

    def pipe_body(it, _):
      g0 = it * 2
      fire(g0 + 1, 1)
      drain(0)
      compute(g0, 0)

      @pl.when(g0 + 2 < _NG)
      def _fire_next():
        fire(g0 + 2, 0)

      drain(1)
      compute(g0 + 1, 1)
      return _

    lax.fori_loop(0, _NG // 2, pipe_body, 0)

    pltpu.sync_copy(pos_v, pos_hbm.at[pl.ds(base, _BW)])
    pltpu.sync_copy(neg_v, neg_hbm.at[pl.ds(nbase, _BW * _K)])

  return body(center_ids, context_ids, neg_ids_flat, t4c, t4x)


def _finish_body(pos_ref, neg_ref, out_ref):
  p = jnp.clip(pos_ref[...], -_CLIP, _CLIP)
  n = jnp.clip(neg_ref[...], -_CLIP, _CLIP)
  lp = jnp.maximum(-p, 0.0) + jnp.log1p(jnp.exp(-jnp.abs(p)))
  ln = jnp.maximum(n, 0.0) + jnp.log1p(jnp.exp(-jnp.abs(n)))
  total = jnp.sum(lp) + jnp.sum(ln)
  out_ref[...] = jnp.reshape(total * (1.0 / _B), (1, 1))


def kernel(center_ids, context_ids, neg_context_ids, center_emb, context_emb):
  cid = center_ids.astype(jnp.int32)
  xid = context_ids.astype(jnp.int32)
  nid = neg_context_ids.astype(jnp.int32).reshape(-1)
  t4c = center_emb.reshape(_V // 4, 4 * _D)
  t4x = context_emb.reshape(_V // 4, 4 * _D)
  pos, neg = _sc_scores(cid, xid, nid, t4c, t4x)
  out = pl.pallas_call(
      _finish_body,
      out_shape=jax.ShapeDtypeStruct((1, 1), jnp.float32),
  )(pos.reshape(_B // 128, 128), neg.reshape(_B * _K // 128, 128))
  return out[0, 0]


# speedup vs baseline: 1.4826x; 1.0059x over previous
import functools

import jax
import jax.numpy as jnp
from jax import lax
from jax.experimental import pallas as pl
from jax.experimental.pallas import tpu as pltpu
from jax.experimental.pallas import tpu_sc as plsc

_B = 16384
_D = 32
_K = 5
_CLIP = 10.0

_NC = 2
_NS = 16
_L = 16
_NW = _NC * _NS
_BW = _B // _NW
_CH = 128


def _sc_scores(center_ids, context_ids, neg_ids_flat, center_emb, context_emb):
  mesh = plsc.VectorSubcoreMesh(core_axis_name="c", subcore_axis_name="s")

  @functools.partial(
      pl.kernel,
      mesh=mesh,
      compiler_params=pltpu.CompilerParams(
          needs_layout_passes=False, use_tc_tiling_on_sc=False),
      out_type=[
          jax.ShapeDtypeStruct((_B,), jnp.float32),
          jax.ShapeDtypeStruct((_B * _K,), jnp.float32),
      ],
      scratch_types=[
          pltpu.VMEM((_BW,), jnp.int32),
          pltpu.VMEM((_BW,), jnp.int32),
          pltpu.VMEM((_BW * _K,), jnp.int32),
          pltpu.VMEM((_BW, _D), jnp.float32),
          pltpu.VMEM((_BW, _D), jnp.float32),
          pltpu.VMEM((_BW * _K, _D), jnp.float32),
          pltpu.VMEM((_BW,), jnp.float32),
          pltpu.VMEM((_BW * _K,), jnp.float32),
          pltpu.SemaphoreType.DMA,
      ],
  )
  def body(cid_hbm, xid_hbm, nid_hbm, cemb_hbm, xemb_hbm,
           pos_hbm, neg_hbm,
           idx_c, idx_x, idx_n, rows_c, rows_x, rows_n, pos_v, neg_v, sem):
    wid = lax.axis_index("s") * _NC + lax.axis_index("c")
    base = wid * _BW
    nbase = wid * (_BW * _K)

    pltpu.sync_copy(cid_hbm.at[pl.ds(base, _BW)], idx_c)
    pltpu.sync_copy(xid_hbm.at[pl.ds(base, _BW)], idx_x)
    pltpu.sync_copy(nid_hbm.at[pl.ds(nbase, _BW * _K)], idx_n)

    copies = []
    for j in range(_BW // _CH):
      sl = pl.ds(j * _CH, _CH)
      copies.append(pltpu.async_copy(cemb_hbm.at[idx_c.at[sl]], rows_c.at[sl], sem))
      copies.append(pltpu.async_copy(xemb_hbm.at[idx_x.at[sl]], rows_x.at[sl], sem))
    for j in range(_BW * _K // _CH):
      sl = pl.ds(j * _CH, _CH)
      copies.append(pltpu.async_copy(xemb_hbm.at[idx_n.at[sl]], rows_n.at[sl], sem))
    for c in copies:
      c.wait()

    iota = lax.iota(jnp.int32, _L)

    def group(g, carry):
      b_vec = g * _L + iota
      b5 = b_vec * _K
      acc_p = jnp.zeros((_L,), jnp.float32)
      accs = [jnp.zeros((_L,), jnp.float32) for _ in range(_K)]
      for d in range(_D):
        dcol = jnp.full((_L,), d, jnp.int32)
        c_col = plsc.load_gather(rows_c, [b_vec, dcol])
        x_col = plsc.load_gather(rows_x, [b_vec, dcol])
        acc_p = acc_p + c_col * x_col
        for k in range(_K):
          n_col = plsc.load_gather(rows_n, [b5 + k, dcol])
          accs[k] = accs[k] + c_col * n_col
      pos_v[pl.ds(g * _L, _L)] = acc_p
      for k in range(_K):
        neg_v[pl.ds(k * _BW + g * _L, _L)] = accs[k]
      return carry

    lax.fori_loop(0, _BW // _L, group, 0)

    pltpu.sync_copy(pos_v, pos_hbm.at[pl.ds(base, _BW)])
    pltpu.sync_copy(neg_v, neg_hbm.at[pl.ds(nbase, _BW * _K)])

  return body(center_ids, context_ids, neg_ids_flat, center_emb, context_emb)


def _finish_body(pos_ref, neg_ref, out_ref):
  p = jnp.clip(pos_ref[...], -_CLIP, _CLIP)
  n = jnp.clip(neg_ref[...], -_CLIP, _CLIP)
  lp = jnp.maximum(-p, 0.0) + jnp.log1p(jnp.exp(-jnp.abs(p)))
  ln = jnp.maximum(n, 0.0) + jnp.log1p(jnp.exp(-jnp.abs(n)))
  total = jnp.sum(lp) + jnp.sum(ln)
  out_ref[...] = jnp.reshape(total * (1.0 / _B), (1, 1))


def kernel(center_ids, context_ids, neg_context_ids, center_emb, context_emb):
  cid = center_ids.astype(jnp.int32)
  xid = context_ids.astype(jnp.int32)
  nid = neg_context_ids.astype(jnp.int32).reshape(-1)
  pos, neg = _sc_scores(cid, xid, nid, center_emb, context_emb)
  out = pl.pallas_call(
      _finish_body,
      out_shape=jax.ShapeDtypeStruct((1, 1), jnp.float32),
  )(pos.reshape(_B // 128, 128), neg.reshape(_B * _K // 128, 128))
  return out[0, 0]
